# baseline (speedup 1.0000x reference)
import functools

import jax
import jax.numpy as jnp
from jax import lax
from jax.experimental import pallas as pl
from jax.experimental.pallas import tpu as pltpu
from jax.experimental.pallas import tpu_sc as plsc

N = 10000
D = 128
H = 128
C = 32
G = 64

NC = 2
NS = 16
LANE = 16
NW = NC * NS

NBKT = 32
BSIZE = 313
NP = NBKT * BSIZE
ACCROWS = 320
PADROW = BSIZE
HP = H + 16

NB = 1000
NGRID = N // NB

EPS = 1e-16

_SC_PARAMS = pltpu.CompilerParams(
    needs_layout_passes=False, use_tc_tiling_on_sc=False
)


def _tc_first(x, W, a2):
    def body(x_ref, w_ref, a_ref, h_ref, sa_ref, da_ref):
        h = jnp.dot(x_ref[...], w_ref[...], preferred_element_type=jnp.float32)
        h_ref[...] = h
        sada = jnp.dot(h, a_ref[...], preferred_element_type=jnp.float32)
        sa_ref[...] = sada[:, 0:1]
        da_ref[...] = sada[:, 1:2]

    return pl.pallas_call(
        body,
        grid=(NGRID,),
        in_specs=[
            pl.BlockSpec((NB, D), lambda k: (k, 0)),
            pl.BlockSpec((D, H), lambda k: (0, 0)),
            pl.BlockSpec((H, 2), lambda k: (0, 0)),
        ],
        out_specs=[
            pl.BlockSpec((NB, H), lambda k: (k, 0)),
            pl.BlockSpec((NB, 1), lambda k: (k, 0)),
            pl.BlockSpec((NB, 1), lambda k: (k, 0)),
        ],
        out_shape=[
            jax.ShapeDtypeStruct((N, H), jnp.float32),
            jax.ShapeDtypeStruct((N, 1), jnp.float32),
            jax.ShapeDtypeStruct((N, 1), jnp.float32),
        ],
    )(x, W, a2)


def _tc_mid(acc, b2d, W, a2):
    def body(acc_ref, b_ref, w_ref, a_ref, h_ref, sa_ref, da_ref):
        a0 = acc_ref[...]
        num = a0[:, :H]
        den = a0[:, H:H + 1] + EPS
        feat = jnp.maximum(num / den + b_ref[...], 0.0)
        h = jnp.dot(feat, w_ref[...], preferred_element_type=jnp.float32)
        h_ref[...] = h
        sada = jnp.dot(h, a_ref[...], preferred_element_type=jnp.float32)
        sa_ref[...] = sada[:, 0:1]
        da_ref[...] = sada[:, 1:2]

    return pl.pallas_call(
        body,
        grid=(NGRID,),
        in_specs=[
            pl.BlockSpec((NB, HP), lambda k: (k, 0)),
            pl.BlockSpec((1, H), lambda k: (0, 0)),
            pl.BlockSpec((D, H), lambda k: (0, 0)),
            pl.BlockSpec((H, 2), lambda k: (0, 0)),
        ],
        out_specs=[
            pl.BlockSpec((NB, H), lambda k: (k, 0)),
            pl.BlockSpec((NB, 1), lambda k: (k, 0)),
            pl.BlockSpec((NB, 1), lambda k: (k, 0)),
        ],
        out_shape=[
            jax.ShapeDtypeStruct((N, H), jnp.float32),
            jax.ShapeDtypeStruct((N, 1), jnp.float32),
            jax.ShapeDtypeStruct((N, 1), jnp.float32),
        ],
    )(acc, b2d, W, a2)


def _tc_readout(acc, b2d, batch3d, batchm13d, Wp, bp2d):
    def body(acc_ref, b_ref, bat_ref, batm1_ref, wp_ref, bp_ref, out_ref,
             hsel_ref, pres_ref):
        k = pl.program_id(0)

        a0 = acc_ref[...]
        num = a0[:, :H]
        den = a0[:, H:H + 1] + EPS
        h3 = num / den + b_ref[...]

        bat = bat_ref[0]
        batm1 = batm1_ref[0]
        col = lax.broadcasted_iota(jnp.int32, (1, NB), 1) + k * NB
        first = jnp.logical_or(col == 0, bat != batm1)
        gids = lax.broadcasted_iota(jnp.int32, (G, NB), 0)
        onehot = jnp.where(
            jnp.logical_and(bat == gids, first), 1.0, 0.0
        ).astype(jnp.float32)

        part = jnp.dot(onehot, h3, preferred_element_type=jnp.float32)
        pcnt = jnp.sum(onehot, axis=1, keepdims=True)

        @pl.when(k == 0)
        def _():
            hsel_ref[...] = part
            pres_ref[...] = pcnt

        @pl.when(k > 0)
        def _():
            hsel_ref[...] = hsel_ref[...] + part
            pres_ref[...] = pres_ref[...] + pcnt

        @pl.when(k == NGRID - 1)
        def _():
            lastrow = h3[NB - 1:NB, :]
            hsel = hsel_ref[...] + (1.0 - pres_ref[...]) * lastrow
            out_ref[...] = (
                jnp.dot(hsel, wp_ref[...], preferred_element_type=jnp.float32)
                + bp_ref[...]
            )

    return pl.pallas_call(
        body,
        grid=(NGRID,),
        in_specs=[
            pl.BlockSpec((NB, HP), lambda k: (k, 0)),
            pl.BlockSpec((1, H), lambda k: (0, 0)),
            pl.BlockSpec((1, 1, NB), lambda k: (k, 0, 0)),
            pl.BlockSpec((1, 1, NB), lambda k: (k, 0, 0)),
            pl.BlockSpec((H, C), lambda k: (0, 0)),
            pl.BlockSpec((1, C), lambda k: (0, 0)),
        ],
        out_specs=pl.BlockSpec((G, C), lambda k: (0, 0)),
        out_shape=jax.ShapeDtypeStruct((G, C), jnp.float32),
        scratch_shapes=[
            pltpu.VMEM((G, H), jnp.float32),
            pltpu.VMEM((G, 1), jnp.float32),
        ],
    )(acc, b2d, batch3d, batchm13d, Wp, bp2d)


def _sc_bin_edges(src_pad, dst_pad, e_real, e_tot):
    per_tile = e_tot // NW
    n_chunks = per_tile // 128
    mesh = plsc.VectorSubcoreMesh(core_axis_name="c", subcore_axis_name="s")

    @functools.partial(
        pl.kernel,
        out_type=[
            jax.ShapeDtypeStruct((NC, NS, NBKT, per_tile), jnp.int32),
            jax.ShapeDtypeStruct((NC, NS, NBKT, per_tile), jnp.int32),
            jax.ShapeDtypeStruct((NC, NS, NBKT * LANE), jnp.int32),
        ],
        mesh=mesh,
        compiler_params=_SC_PARAMS,
        scratch_types=[
            pltpu.VMEM((128,), jnp.int32),
            pltpu.VMEM((128,), jnp.int32),
            pltpu.VMEM((NBKT, 256), jnp.int32),
            pltpu.VMEM((NBKT, 256), jnp.int32),
            pltpu.VMEM((NBKT * LANE,), jnp.int32),
            pltpu.SMEM((NBKT,), jnp.int32),
            pltpu.SMEM((NBKT,), jnp.int32),
        ],
    )
    def k(src_hbm, dst_hbm, bsrc, bdst, cnts,
          srcb, dstb, ssrc, sdst, cnt_v, scnt, bcnt):
        c = lax.axis_index("c")
        s = lax.axis_index("s")
        wid = c * NS + s
        for b in range(NBKT):
            scnt[b] = 0
            bcnt[b] = 0

        def chunk_body(ch, _):
            base = wid * per_tile + ch * 128
            pltpu.sync_copy(src_hbm.at[pl.ds(base, 128)], srcb)
            pltpu.sync_copy(dst_hbm.at[pl.ds(base, 128)], dstb)

            def group(j, _):
                sl = pl.ds(j * LANE, LANE)
                sv = srcb[sl]
                dv = dstb[sl]
                eid = base + j * LANE + lax.iota(jnp.int32, LANE)
                valid = eid < e_real
                bv = dv // BSIZE
                for b in range(NBKT):
                    m = jnp.logical_and(valid, bv == b)
                    off = scnt[b]
                    plsc.store_compressed(ssrc.at[b, pl.ds(off, LANE)], sv,
                                          mask=m)
                    plsc.store_compressed(sdst.at[b, pl.ds(off, LANE)], dv,
                                          mask=m)
                    npop = plsc.all_reduce_population_count(m)[0]
                    scnt[b] = off + npop
                return 0

            lax.fori_loop(0, 8, group, 0)

            for b in range(NBKT):
                sc = scnt[b]

                @pl.when(sc >= 128)
                def _():
                    blk = bcnt[b]
                    pltpu.sync_copy(ssrc.at[b, pl.ds(0, 128)],
                                    bsrc.at[c, s, b, pl.ds(blk * 128, 128)])
                    pltpu.sync_copy(sdst.at[b, pl.ds(0, 128)],
                                    bdst.at[c, s, b, pl.ds(blk * 128, 128)])
                    for q in range(8):
                        ssrc[b, pl.ds(q * LANE, LANE)] = (
                            ssrc[b, pl.ds(128 + q * LANE, LANE)])
                        sdst[b, pl.ds(q * LANE, LANE)] = (
                            sdst[b, pl.ds(128 + q * LANE, LANE)])
                    scnt[b] = sc - 128
                    bcnt[b] = blk + 1
            return 0

        lax.fori_loop(0, n_chunks, chunk_body, 0)

        for b in range(NBKT):
            sc = scnt[b]
            blk = bcnt[b]

            @pl.when(sc > 0)
            def _():
                pltpu.sync_copy(ssrc.at[b, pl.ds(0, 128)],
                                bsrc.at[c, s, b, pl.ds(blk * 128, 128)])
                pltpu.sync_copy(sdst.at[b, pl.ds(0, 128)],
                                bdst.at[c, s, b, pl.ds(blk * 128, 128)])
            cnt_v[pl.ds(b * LANE, LANE)] = jnp.full(
                (LANE,), blk * 128 + sc, jnp.int32)
        pltpu.sync_copy(cnt_v, cnts.at[c, s])

    return k(src_pad, dst_pad)


def _sc_edge_pass(bsrc, bdst, cnts, h, sa, da, per_tile):
    mesh = plsc.VectorSubcoreMesh(core_axis_name="c", subcore_axis_name="s")

    @functools.partial(
        pl.kernel,
        out_type=jax.ShapeDtypeStruct((NP, HP), jnp.float32),
        mesh=mesh,
        compiler_params=_SC_PARAMS,
        scratch_types=[
            pltpu.VMEM((N,), jnp.float32),
            pltpu.VMEM((N,), jnp.float32),
            pltpu.VMEM((NC, NS, NBKT * LANE), jnp.int32),
            pltpu.VMEM((128,), jnp.int32),
            pltpu.VMEM((128,), jnp.int32),
            pltpu.VMEM((128,), jnp.float32),
            pltpu.VMEM((128, H), jnp.float32),
            pltpu.VMEM((ACCROWS, HP), jnp.float32),
            pltpu.SemaphoreType.DMA,
        ],
    )
    def k(bsrc_hbm, bdst_hbm, cnts_hbm, h_hbm, sa_hbm, da_hbm, out_hbm,
          sa_v, da_v, cntv, srcb, dstb, ev, grow, acc_t, sem):
        c = lax.axis_index("c")
        s = lax.axis_index("s")
        b = c * NS + s
        base_row = b * BSIZE

        pltpu.sync_copy(sa_hbm, sa_v)
        pltpu.sync_copy(da_hbm, da_v)
        pltpu.sync_copy(cnts_hbm, cntv)

        zed = jnp.zeros((LANE,), jnp.float32)

        def zrow(r, _):
            for j in range(HP // LANE):
                acc_t[r, pl.ds(j * LANE, LANE)] = zed
            return 0

        lax.fori_loop(0, ACCROWS, zrow, 0)

        def writer_body(w, _):
            cw, sw = w // NS, w % NS
            cnt = cntv[cw, sw, pl.ds(b * LANE, LANE)][0]
            nblk = (cnt + 127) // 128

            def blk_body(kb, _):
                off = kb * 128
                pltpu.sync_copy(bsrc_hbm.at[cw, sw, b, pl.ds(off, 128)], srcb)
                pltpu.sync_copy(bdst_hbm.at[cw, sw, b, pl.ds(off, 128)], dstb)

                def group(g, _):
                    sl = pl.ds(g * LANE, LANE)
                    dv = dstb[sl]
                    eidx = off + g * LANE + lax.iota(jnp.int32, LANE)
                    m = eidx < cnt
                    sv = jnp.where(m, srcb[sl], 0)
                    srcb[sl] = sv
                    sav = plsc.load_gather(sa_v, [sv])
                    dav = plsc.load_gather(da_v, [jnp.where(m, dv, 0)])
                    xv = sav + dav
                    lv = jnp.where(xv >= 0.0, xv, 0.2 * xv)
                    ev[sl] = jnp.where(m, jnp.exp(lv), 0.0)
                    dstb[sl] = jnp.where(m, dv - base_row, PADROW)
                    return 0

                lax.fori_loop(0, 8, group, 0)
                pltpu.async_copy(h_hbm.at[srcb], grow, sem).wait()

                def accg(g, _):
                    evv = ev[pl.ds(g * LANE, LANE)]
                    dlv = dstb[pl.ds(g * LANE, LANE)]
                    for i in range(LANE):
                        er = jnp.full((LANE,), evv[i], jnp.float32)
                        dl = dlv[i]
                        r = g * LANE + i
                        for j in range(H // LANE):
                            plsc.addupdate(
                                acc_t.at[dl, pl.ds(j * LANE, LANE)],
                                grow[r, pl.ds(j * LANE, LANE)] * er,
                            )
                        plsc.addupdate(acc_t.at[dl, pl.ds(H, LANE)], er)
                    return 0

                lax.fori_loop(0, 8, accg, 0)
                return 0

            lax.fori_loop(0, nblk, blk_body, 0)
            return 0

        lax.fori_loop(0, NW, writer_body, 0)

        pltpu.sync_copy(acc_t.at[pl.ds(0, BSIZE)],
                        out_hbm.at[pl.ds(base_row, BSIZE)])

    return k(bsrc, bdst, cnts, h, sa, da)


def kernel(x, edge_index, batch, W1, as1, ad1, b1, W2, as2, ad2, b2,
           W3, as3, ad3, b3, Wp, bp):
    E = edge_index.shape[1]
    e_real = E + N
    e_tot = ((e_real + NW * 128 - 1) // (NW * 128)) * (NW * 128)
    pad = e_tot - e_real
    per_tile = e_tot // NW

    loops = jnp.arange(N, dtype=jnp.int32)
    zpad = jnp.zeros((pad,), jnp.int32)
    src = jnp.concatenate([edge_index[0].astype(jnp.int32), loops, zpad])
    dst = jnp.concatenate([edge_index[1].astype(jnp.int32), loops, zpad])

    bsrc, bdst, cnts = _sc_bin_edges(src, dst, e_real, e_tot)

    a21 = jnp.stack([as1, ad1], axis=1)
    a22 = jnp.stack([as2, ad2], axis=1)
    a23 = jnp.stack([as3, ad3], axis=1)

    h1, sa1, da1 = _tc_first(x.astype(jnp.float32), W1, a21)
    acc1 = _sc_edge_pass(bsrc, bdst, cnts, h1, sa1.reshape(N), da1.reshape(N),
                         per_tile)
    h2, sa2, da2 = _tc_mid(acc1, b1.reshape(1, H), W2, a22)
    acc2 = _sc_edge_pass(bsrc, bdst, cnts, h2, sa2.reshape(N), da2.reshape(N),
                         per_tile)
    h3, sa3, da3 = _tc_mid(acc2, b2.reshape(1, H), W3, a23)
    acc3 = _sc_edge_pass(bsrc, bdst, cnts, h3, sa3.reshape(N), da3.reshape(N),
                         per_tile)

    batf = batch.astype(jnp.int32)
    batm1f = jnp.concatenate([batf[:1], batf[:-1]])
    batch3d = batf.reshape(NGRID, 1, NB)
    batchm13d = batm1f.reshape(NGRID, 1, NB)
    logits = _tc_readout(acc3, b3.reshape(1, H), batch3d, batchm13d,
                         Wp, bp.reshape(1, C))
    return logits


# speedup vs baseline: 1.0076x; 1.0076x over previous
import functools

import jax
import jax.numpy as jnp
from jax import lax
from jax.experimental import pallas as pl
from jax.experimental.pallas import tpu as pltpu
from jax.experimental.pallas import tpu_sc as plsc

N = 10000
D = 128
H = 128
C = 32
G = 64

NC = 2
NS = 16
LANE = 16
NW = NC * NS

NBKT = 32
BSIZE = 313
NP = NBKT * BSIZE
ACCROWS = 320
PADROW = BSIZE
HP = H + 16

NB = 1000
NGRID = N // NB

EPS = 1e-16

_SC_PARAMS = pltpu.CompilerParams(
    needs_layout_passes=False, use_tc_tiling_on_sc=False
)


def _tc_first(x, W, a2):
    def body(x_ref, w_ref, a_ref, h_ref, sa_ref, da_ref):
        h = jnp.dot(x_ref[...], w_ref[...], preferred_element_type=jnp.float32)
        h_ref[...] = h
        sada = jnp.dot(h, a_ref[...], preferred_element_type=jnp.float32)
        sa_ref[...] = sada[:, 0:1]
        da_ref[...] = sada[:, 1:2]

    return pl.pallas_call(
        body,
        grid=(NGRID,),
        in_specs=[
            pl.BlockSpec((NB, D), lambda k: (k, 0)),
            pl.BlockSpec((D, H), lambda k: (0, 0)),
            pl.BlockSpec((H, 2), lambda k: (0, 0)),
        ],
        out_specs=[
            pl.BlockSpec((NB, H), lambda k: (k, 0)),
            pl.BlockSpec((NB, 1), lambda k: (k, 0)),
            pl.BlockSpec((NB, 1), lambda k: (k, 0)),
        ],
        out_shape=[
            jax.ShapeDtypeStruct((N, H), jnp.float32),
            jax.ShapeDtypeStruct((N, 1), jnp.float32),
            jax.ShapeDtypeStruct((N, 1), jnp.float32),
        ],
    )(x, W, a2)


def _tc_mid(acc, b2d, W, a2):
    def body(acc_ref, b_ref, w_ref, a_ref, h_ref, sa_ref, da_ref):
        a0 = acc_ref[...]
        num = a0[:, :H]
        den = a0[:, H:H + 1] + EPS
        feat = jnp.maximum(num / den + b_ref[...], 0.0)
        h = jnp.dot(feat, w_ref[...], preferred_element_type=jnp.float32)
        h_ref[...] = h
        sada = jnp.dot(h, a_ref[...], preferred_element_type=jnp.float32)
        sa_ref[...] = sada[:, 0:1]
        da_ref[...] = sada[:, 1:2]

    return pl.pallas_call(
        body,
        grid=(NGRID,),
        in_specs=[
            pl.BlockSpec((NB, HP), lambda k: (k, 0)),
            pl.BlockSpec((1, H), lambda k: (0, 0)),
            pl.BlockSpec((D, H), lambda k: (0, 0)),
            pl.BlockSpec((H, 2), lambda k: (0, 0)),
        ],
        out_specs=[
            pl.BlockSpec((NB, H), lambda k: (k, 0)),
            pl.BlockSpec((NB, 1), lambda k: (k, 0)),
            pl.BlockSpec((NB, 1), lambda k: (k, 0)),
        ],
        out_shape=[
            jax.ShapeDtypeStruct((N, H), jnp.float32),
            jax.ShapeDtypeStruct((N, 1), jnp.float32),
            jax.ShapeDtypeStruct((N, 1), jnp.float32),
        ],
    )(acc, b2d, W, a2)


def _tc_readout(acc, b2d, batch3d, batchm13d, Wp, bp2d):
    def body(acc_ref, b_ref, bat_ref, batm1_ref, wp_ref, bp_ref, out_ref,
             hsel_ref, pres_ref):
        k = pl.program_id(0)

        a0 = acc_ref[...]
        num = a0[:, :H]
        den = a0[:, H:H + 1] + EPS
        h3 = num / den + b_ref[...]

        bat = bat_ref[0]
        batm1 = batm1_ref[0]
        col = lax.broadcasted_iota(jnp.int32, (1, NB), 1) + k * NB
        first = jnp.logical_or(col == 0, bat != batm1)
        gids = lax.broadcasted_iota(jnp.int32, (G, NB), 0)
        onehot = jnp.where(
            jnp.logical_and(bat == gids, first), 1.0, 0.0
        ).astype(jnp.float32)

        part = jnp.dot(onehot, h3, preferred_element_type=jnp.float32)
        pcnt = jnp.sum(onehot, axis=1, keepdims=True)

        @pl.when(k == 0)
        def _():
            hsel_ref[...] = part
            pres_ref[...] = pcnt

        @pl.when(k > 0)
        def _():
            hsel_ref[...] = hsel_ref[...] + part
            pres_ref[...] = pres_ref[...] + pcnt

        @pl.when(k == NGRID - 1)
        def _():
            lastrow = h3[NB - 1:NB, :]
            hsel = hsel_ref[...] + (1.0 - pres_ref[...]) * lastrow
            out_ref[...] = (
                jnp.dot(hsel, wp_ref[...], preferred_element_type=jnp.float32)
                + bp_ref[...]
            )

    return pl.pallas_call(
        body,
        grid=(NGRID,),
        in_specs=[
            pl.BlockSpec((NB, HP), lambda k: (k, 0)),
            pl.BlockSpec((1, H), lambda k: (0, 0)),
            pl.BlockSpec((1, 1, NB), lambda k: (k, 0, 0)),
            pl.BlockSpec((1, 1, NB), lambda k: (k, 0, 0)),
            pl.BlockSpec((H, C), lambda k: (0, 0)),
            pl.BlockSpec((1, C), lambda k: (0, 0)),
        ],
        out_specs=pl.BlockSpec((G, C), lambda k: (0, 0)),
        out_shape=jax.ShapeDtypeStruct((G, C), jnp.float32),
        scratch_shapes=[
            pltpu.VMEM((G, H), jnp.float32),
            pltpu.VMEM((G, 1), jnp.float32),
        ],
    )(acc, b2d, batch3d, batchm13d, Wp, bp2d)


def _sc_bin_edges(src_pad, dst_pad, e_real, e_tot):
    per_tile = e_tot // NW
    n_chunks = per_tile // 128
    mesh = plsc.VectorSubcoreMesh(core_axis_name="c", subcore_axis_name="s")

    @functools.partial(
        pl.kernel,
        out_type=[
            jax.ShapeDtypeStruct((NC, NS, NBKT, per_tile), jnp.int32),
            jax.ShapeDtypeStruct((NC, NS, NBKT, per_tile), jnp.int32),
            jax.ShapeDtypeStruct((NC, NS, NBKT * LANE), jnp.int32),
        ],
        mesh=mesh,
        compiler_params=_SC_PARAMS,
        scratch_types=[
            pltpu.VMEM((128,), jnp.int32),
            pltpu.VMEM((128,), jnp.int32),
            pltpu.VMEM((NBKT, 256), jnp.int32),
            pltpu.VMEM((NBKT, 256), jnp.int32),
            pltpu.VMEM((NBKT * LANE,), jnp.int32),
            pltpu.SMEM((NBKT,), jnp.int32),
            pltpu.SMEM((NBKT,), jnp.int32),
        ],
    )
    def k(src_hbm, dst_hbm, bsrc, bdst, cnts,
          srcb, dstb, ssrc, sdst, cnt_v, scnt, bcnt):
        c = lax.axis_index("c")
        s = lax.axis_index("s")
        wid = c * NS + s
        for b in range(NBKT):
            scnt[b] = 0
            bcnt[b] = 0

        def chunk_body(ch, _):
            base = wid * per_tile + ch * 128
            pltpu.sync_copy(src_hbm.at[pl.ds(base, 128)], srcb)
            pltpu.sync_copy(dst_hbm.at[pl.ds(base, 128)], dstb)

            def group(j, _):
                sl = pl.ds(j * LANE, LANE)
                sv = srcb[sl]
                dv = dstb[sl]
                eid = base + j * LANE + lax.iota(jnp.int32, LANE)
                valid = eid < e_real
                bv = dv // BSIZE
                for b in range(NBKT):
                    m = jnp.logical_and(valid, bv == b)
                    off = scnt[b]
                    plsc.store_compressed(ssrc.at[b, pl.ds(off, LANE)], sv,
                                          mask=m)
                    plsc.store_compressed(sdst.at[b, pl.ds(off, LANE)], dv,
                                          mask=m)
                    npop = plsc.all_reduce_population_count(m)[0]
                    scnt[b] = off + npop
                return 0

            lax.fori_loop(0, 8, group, 0)

            for b in range(NBKT):
                sc = scnt[b]

                @pl.when(sc >= 128)
                def _():
                    blk = bcnt[b]
                    pltpu.sync_copy(ssrc.at[b, pl.ds(0, 128)],
                                    bsrc.at[c, s, b, pl.ds(blk * 128, 128)])
                    pltpu.sync_copy(sdst.at[b, pl.ds(0, 128)],
                                    bdst.at[c, s, b, pl.ds(blk * 128, 128)])
                    for q in range(8):
                        ssrc[b, pl.ds(q * LANE, LANE)] = (
                            ssrc[b, pl.ds(128 + q * LANE, LANE)])
                        sdst[b, pl.ds(q * LANE, LANE)] = (
                            sdst[b, pl.ds(128 + q * LANE, LANE)])
                    scnt[b] = sc - 128
                    bcnt[b] = blk + 1
            return 0

        lax.fori_loop(0, n_chunks, chunk_body, 0)

        for b in range(NBKT):
            sc = scnt[b]
            blk = bcnt[b]

            @pl.when(sc > 0)
            def _():
                pltpu.sync_copy(ssrc.at[b, pl.ds(0, 128)],
                                bsrc.at[c, s, b, pl.ds(blk * 128, 128)])
                pltpu.sync_copy(sdst.at[b, pl.ds(0, 128)],
                                bdst.at[c, s, b, pl.ds(blk * 128, 128)])
            cnt_v[pl.ds(b * LANE, LANE)] = jnp.full(
                (LANE,), blk * 128 + sc, jnp.int32)
        pltpu.sync_copy(cnt_v, cnts.at[c, s])

    return k(src_pad, dst_pad)


def _sc_edge_pass(bsrc, bdst, cnts, h, sa, da, per_tile):
    mesh = plsc.VectorSubcoreMesh(core_axis_name="c", subcore_axis_name="s")

    @functools.partial(
        pl.kernel,
        out_type=jax.ShapeDtypeStruct((NP, HP), jnp.float32),
        mesh=mesh,
        compiler_params=_SC_PARAMS,
        scratch_types=[
            pltpu.VMEM((N,), jnp.float32),
            pltpu.VMEM((N,), jnp.float32),
            pltpu.VMEM((NC, NS, NBKT * LANE), jnp.int32),
            pltpu.VMEM((128,), jnp.int32),
            pltpu.VMEM((128,), jnp.int32),
            pltpu.VMEM((128,), jnp.float32),
            pltpu.VMEM((128, H), jnp.float32),
            pltpu.VMEM((ACCROWS, HP), jnp.float32),
            pltpu.SemaphoreType.DMA,
        ],
    )
    def k(bsrc_hbm, bdst_hbm, cnts_hbm, h_hbm, sa_hbm, da_hbm, out_hbm,
          sa_v, da_v, cntv, srcb, dstb, ev, grow, acc_t, sem):
        c = lax.axis_index("c")
        s = lax.axis_index("s")
        b = c * NS + s
        base_row = b * BSIZE

        pltpu.sync_copy(sa_hbm, sa_v)
        pltpu.sync_copy(da_hbm, da_v)
        pltpu.sync_copy(cnts_hbm, cntv)

        zed = jnp.zeros((LANE,), jnp.float32)

        @plsc.parallel_loop(0, ACCROWS)
        def zrow(r):
            for j in range(HP // LANE):
                acc_t[r, pl.ds(j * LANE, LANE)] = zed

        def writer_body(w, _):
            cw, sw = w // NS, w % NS
            cnt = cntv[cw, sw, pl.ds(b * LANE, LANE)][0]
            nblk = (cnt + 127) // 128

            def blk_body(kb, _):
                off = kb * 128
                pltpu.sync_copy(bsrc_hbm.at[cw, sw, b, pl.ds(off, 128)], srcb)
                pltpu.sync_copy(bdst_hbm.at[cw, sw, b, pl.ds(off, 128)], dstb)

                @plsc.parallel_loop(0, 8)
                def group(g):
                    sl = pl.ds(g * LANE, LANE)
                    dv = dstb[sl]
                    eidx = off + g * LANE + lax.iota(jnp.int32, LANE)
                    m = eidx < cnt
                    sv = jnp.where(m, srcb[sl], 0)
                    srcb[sl] = sv
                    sav = plsc.load_gather(sa_v, [sv])
                    dav = plsc.load_gather(da_v, [jnp.where(m, dv, 0)])
                    xv = sav + dav
                    lv = jnp.where(xv >= 0.0, xv, 0.2 * xv)
                    ev[sl] = jnp.where(m, jnp.exp(lv), 0.0)
                    dstb[sl] = jnp.where(m, dv - base_row, PADROW)
                pltpu.async_copy(h_hbm.at[srcb], grow, sem).wait()

                @plsc.parallel_loop(0, 8)
                def accg(g):
                    evv = ev[pl.ds(g * LANE, LANE)]
                    dlv = dstb[pl.ds(g * LANE, LANE)]
                    for i in range(LANE):
                        er = jnp.full((LANE,), evv[i], jnp.float32)
                        dl = dlv[i]
                        r = g * LANE + i
                        for j in range(H // LANE):
                            plsc.addupdate(
                                acc_t.at[dl, pl.ds(j * LANE, LANE)],
                                grow[r, pl.ds(j * LANE, LANE)] * er,
                            )
                        plsc.addupdate(acc_t.at[dl, pl.ds(H, LANE)], er)
                return 0

            lax.fori_loop(0, nblk, blk_body, 0)
            return 0

        lax.fori_loop(0, NW, writer_body, 0)

        pltpu.sync_copy(acc_t.at[pl.ds(0, BSIZE)],
                        out_hbm.at[pl.ds(base_row, BSIZE)])

    return k(bsrc, bdst, cnts, h, sa, da)


def kernel(x, edge_index, batch, W1, as1, ad1, b1, W2, as2, ad2, b2,
           W3, as3, ad3, b3, Wp, bp):
    E = edge_index.shape[1]
    e_real = E + N
    e_tot = ((e_real + NW * 128 - 1) // (NW * 128)) * (NW * 128)
    pad = e_tot - e_real
    per_tile = e_tot // NW

    loops = jnp.arange(N, dtype=jnp.int32)
    zpad = jnp.zeros((pad,), jnp.int32)
    src = jnp.concatenate([edge_index[0].astype(jnp.int32), loops, zpad])
    dst = jnp.concatenate([edge_index[1].astype(jnp.int32), loops, zpad])

    bsrc, bdst, cnts = _sc_bin_edges(src, dst, e_real, e_tot)

    a21 = jnp.stack([as1, ad1], axis=1)
    a22 = jnp.stack([as2, ad2], axis=1)
    a23 = jnp.stack([as3, ad3], axis=1)

    h1, sa1, da1 = _tc_first(x.astype(jnp.float32), W1, a21)
    acc1 = _sc_edge_pass(bsrc, bdst, cnts, h1, sa1.reshape(N), da1.reshape(N),
                         per_tile)
    h2, sa2, da2 = _tc_mid(acc1, b1.reshape(1, H), W2, a22)
    acc2 = _sc_edge_pass(bsrc, bdst, cnts, h2, sa2.reshape(N), da2.reshape(N),
                         per_tile)
    h3, sa3, da3 = _tc_mid(acc2, b2.reshape(1, H), W3, a23)
    acc3 = _sc_edge_pass(bsrc, bdst, cnts, h3, sa3.reshape(N), da3.reshape(N),
                         per_tile)

    batf = batch.astype(jnp.int32)
    batm1f = jnp.concatenate([batf[:1], batf[:-1]])
    batch3d = batf.reshape(NGRID, 1, NB)
    batchm13d = batm1f.reshape(NGRID, 1, NB)
    logits = _tc_readout(acc3, b3.reshape(1, H), batch3d, batchm13d,
                         Wp, bp.reshape(1, C))
    return logits


# speedup vs baseline: 1.0100x; 1.0023x over previous
import functools

import jax
import jax.numpy as jnp
from jax import lax
from jax.experimental import pallas as pl
from jax.experimental.pallas import tpu as pltpu
from jax.experimental.pallas import tpu_sc as plsc

N = 10000
D = 128
H = 128
C = 32
G = 64

NC = 2
NS = 16
LANE = 16
NW = NC * NS

NBKT = 32
BSIZE = 313
NP = NBKT * BSIZE
ACCROWS = 320
PADROW = BSIZE
HP = H + 16

NB = 1000
NGRID = N // NB

EPS = 1e-16

_SC_PARAMS = pltpu.CompilerParams(
    needs_layout_passes=False, use_tc_tiling_on_sc=False
)


def _tc_first(x, W, a2):
    def body(x_ref, w_ref, a_ref, h_ref, sa_ref, da_ref):
        h = jnp.dot(x_ref[...], w_ref[...], preferred_element_type=jnp.float32)
        h_ref[...] = h
        sada = jnp.dot(h, a_ref[...], preferred_element_type=jnp.float32)
        sa_ref[...] = sada[:, 0:1]
        da_ref[...] = sada[:, 1:2]

    return pl.pallas_call(
        body,
        grid=(NGRID,),
        in_specs=[
            pl.BlockSpec((NB, D), lambda k: (k, 0)),
            pl.BlockSpec((D, H), lambda k: (0, 0)),
            pl.BlockSpec((H, 2), lambda k: (0, 0)),
        ],
        out_specs=[
            pl.BlockSpec((NB, H), lambda k: (k, 0)),
            pl.BlockSpec((NB, 1), lambda k: (k, 0)),
            pl.BlockSpec((NB, 1), lambda k: (k, 0)),
        ],
        out_shape=[
            jax.ShapeDtypeStruct((N, H), jnp.float32),
            jax.ShapeDtypeStruct((N, 1), jnp.float32),
            jax.ShapeDtypeStruct((N, 1), jnp.float32),
        ],
    )(x, W, a2)


def _tc_mid(acc, b2d, W, a2):
    def body(acc_ref, b_ref, w_ref, a_ref, h_ref, sa_ref, da_ref):
        a0 = acc_ref[...]
        num = a0[:, :H]
        den = a0[:, H:H + 1] + EPS
        feat = jnp.maximum(num / den + b_ref[...], 0.0)
        h = jnp.dot(feat, w_ref[...], preferred_element_type=jnp.float32)
        h_ref[...] = h
        sada = jnp.dot(h, a_ref[...], preferred_element_type=jnp.float32)
        sa_ref[...] = sada[:, 0:1]
        da_ref[...] = sada[:, 1:2]

    return pl.pallas_call(
        body,
        grid=(NGRID,),
        in_specs=[
            pl.BlockSpec((NB, HP), lambda k: (k, 0)),
            pl.BlockSpec((1, H), lambda k: (0, 0)),
            pl.BlockSpec((D, H), lambda k: (0, 0)),
            pl.BlockSpec((H, 2), lambda k: (0, 0)),
        ],
        out_specs=[
            pl.BlockSpec((NB, H), lambda k: (k, 0)),
            pl.BlockSpec((NB, 1), lambda k: (k, 0)),
            pl.BlockSpec((NB, 1), lambda k: (k, 0)),
        ],
        out_shape=[
            jax.ShapeDtypeStruct((N, H), jnp.float32),
            jax.ShapeDtypeStruct((N, 1), jnp.float32),
            jax.ShapeDtypeStruct((N, 1), jnp.float32),
        ],
    )(acc, b2d, W, a2)


def _tc_readout(acc, b2d, batch3d, batchm13d, Wp, bp2d):
    def body(acc_ref, b_ref, bat_ref, batm1_ref, wp_ref, bp_ref, out_ref,
             hsel_ref, pres_ref):
        k = pl.program_id(0)

        a0 = acc_ref[...]
        num = a0[:, :H]
        den = a0[:, H:H + 1] + EPS
        h3 = num / den + b_ref[...]

        bat = bat_ref[0]
        batm1 = batm1_ref[0]
        col = lax.broadcasted_iota(jnp.int32, (1, NB), 1) + k * NB
        first = jnp.logical_or(col == 0, bat != batm1)
        gids = lax.broadcasted_iota(jnp.int32, (G, NB), 0)
        onehot = jnp.where(
            jnp.logical_and(bat == gids, first), 1.0, 0.0
        ).astype(jnp.float32)

        part = jnp.dot(onehot, h3, preferred_element_type=jnp.float32)
        pcnt = jnp.sum(onehot, axis=1, keepdims=True)

        @pl.when(k == 0)
        def _():
            hsel_ref[...] = part
            pres_ref[...] = pcnt

        @pl.when(k > 0)
        def _():
            hsel_ref[...] = hsel_ref[...] + part
            pres_ref[...] = pres_ref[...] + pcnt

        @pl.when(k == NGRID - 1)
        def _():
            lastrow = h3[NB - 1:NB, :]
            hsel = hsel_ref[...] + (1.0 - pres_ref[...]) * lastrow
            out_ref[...] = (
                jnp.dot(hsel, wp_ref[...], preferred_element_type=jnp.float32)
                + bp_ref[...]
            )

    return pl.pallas_call(
        body,
        grid=(NGRID,),
        in_specs=[
            pl.BlockSpec((NB, HP), lambda k: (k, 0)),
            pl.BlockSpec((1, H), lambda k: (0, 0)),
            pl.BlockSpec((1, 1, NB), lambda k: (k, 0, 0)),
            pl.BlockSpec((1, 1, NB), lambda k: (k, 0, 0)),
            pl.BlockSpec((H, C), lambda k: (0, 0)),
            pl.BlockSpec((1, C), lambda k: (0, 0)),
        ],
        out_specs=pl.BlockSpec((G, C), lambda k: (0, 0)),
        out_shape=jax.ShapeDtypeStruct((G, C), jnp.float32),
        scratch_shapes=[
            pltpu.VMEM((G, H), jnp.float32),
            pltpu.VMEM((G, 1), jnp.float32),
        ],
    )(acc, b2d, batch3d, batchm13d, Wp, bp2d)


def _sc_bin_edges(src_pad, dst_pad, e_real, e_tot):
    per_tile = e_tot // NW
    n_chunks = per_tile // 128
    mesh = plsc.VectorSubcoreMesh(core_axis_name="c", subcore_axis_name="s")

    @functools.partial(
        pl.kernel,
        out_type=[
            jax.ShapeDtypeStruct((NC, NS, NBKT, per_tile), jnp.int32),
            jax.ShapeDtypeStruct((NC, NS, NBKT, per_tile), jnp.int32),
            jax.ShapeDtypeStruct((NC, NS, NBKT * LANE), jnp.int32),
        ],
        mesh=mesh,
        compiler_params=_SC_PARAMS,
        scratch_types=[
            pltpu.VMEM((128,), jnp.int32),
            pltpu.VMEM((128,), jnp.int32),
            pltpu.VMEM((NBKT, 256), jnp.int32),
            pltpu.VMEM((NBKT, 256), jnp.int32),
            pltpu.VMEM((NBKT * LANE,), jnp.int32),
            pltpu.SMEM((NBKT,), jnp.int32),
            pltpu.SMEM((NBKT,), jnp.int32),
        ],
    )
    def k(src_hbm, dst_hbm, bsrc, bdst, cnts,
          srcb, dstb, ssrc, sdst, cnt_v, scnt, bcnt):
        c = lax.axis_index("c")
        s = lax.axis_index("s")
        wid = c * NS + s
        for b in range(NBKT):
            scnt[b] = 0
            bcnt[b] = 0

        def chunk_body(ch, _):
            base = wid * per_tile + ch * 128
            pltpu.sync_copy(src_hbm.at[pl.ds(base, 128)], srcb)
            pltpu.sync_copy(dst_hbm.at[pl.ds(base, 128)], dstb)

            def group(j, _):
                sl = pl.ds(j * LANE, LANE)
                sv = srcb[sl]
                dv = dstb[sl]
                eid = base + j * LANE + lax.iota(jnp.int32, LANE)
                valid = eid < e_real
                bv = dv // BSIZE
                for b in range(NBKT):
                    m = jnp.logical_and(valid, bv == b)
                    off = scnt[b]
                    plsc.store_compressed(ssrc.at[b, pl.ds(off, LANE)], sv,
                                          mask=m)
                    plsc.store_compressed(sdst.at[b, pl.ds(off, LANE)], dv,
                                          mask=m)
                    npop = plsc.all_reduce_population_count(m)[0]
                    scnt[b] = off + npop
                return 0

            lax.fori_loop(0, 8, group, 0)

            for b in range(NBKT):
                sc = scnt[b]

                @pl.when(sc >= 128)
                def _():
                    blk = bcnt[b]
                    pltpu.sync_copy(ssrc.at[b, pl.ds(0, 128)],
                                    bsrc.at[c, s, b, pl.ds(blk * 128, 128)])
                    pltpu.sync_copy(sdst.at[b, pl.ds(0, 128)],
                                    bdst.at[c, s, b, pl.ds(blk * 128, 128)])
                    for q in range(8):
                        ssrc[b, pl.ds(q * LANE, LANE)] = (
                            ssrc[b, pl.ds(128 + q * LANE, LANE)])
                        sdst[b, pl.ds(q * LANE, LANE)] = (
                            sdst[b, pl.ds(128 + q * LANE, LANE)])
                    scnt[b] = sc - 128
                    bcnt[b] = blk + 1
            return 0

        lax.fori_loop(0, n_chunks, chunk_body, 0)

        for b in range(NBKT):
            sc = scnt[b]
            blk = bcnt[b]

            @pl.when(sc > 0)
            def _():
                pltpu.sync_copy(ssrc.at[b, pl.ds(0, 128)],
                                bsrc.at[c, s, b, pl.ds(blk * 128, 128)])
                pltpu.sync_copy(sdst.at[b, pl.ds(0, 128)],
                                bdst.at[c, s, b, pl.ds(blk * 128, 128)])
            cnt_v[pl.ds(b * LANE, LANE)] = jnp.full(
                (LANE,), blk * 128 + sc, jnp.int32)
        pltpu.sync_copy(cnt_v, cnts.at[c, s])

    return k(src_pad, dst_pad)


def _sc_edge_pass(bsrc, bdst, cnts, h, sa, da, per_tile):
    mesh = plsc.VectorSubcoreMesh(core_axis_name="c", subcore_axis_name="s")

    @functools.partial(
        pl.kernel,
        out_type=jax.ShapeDtypeStruct((NP, HP), jnp.float32),
        mesh=mesh,
        compiler_params=_SC_PARAMS,
        scratch_types=[
            pltpu.VMEM((N,), jnp.float32),
            pltpu.VMEM((N,), jnp.float32),
            pltpu.VMEM((NC, NS, NBKT * LANE), jnp.int32),
            pltpu.VMEM((128,), jnp.int32),
            pltpu.VMEM((128,), jnp.int32),
            pltpu.VMEM((128,), jnp.float32),
            pltpu.VMEM((128, H), jnp.float32),
            pltpu.VMEM((ACCROWS, HP), jnp.float32),
            pltpu.SemaphoreType.DMA,
        ],
    )
    def k(bsrc_hbm, bdst_hbm, cnts_hbm, h_hbm, sa_hbm, da_hbm, out_hbm,
          sa_v, da_v, cntv, srcb, dstb, ev, grow, acc_t, sem):
        c = lax.axis_index("c")
        s = lax.axis_index("s")
        b = c * NS + s
        base_row = b * BSIZE

        pltpu.sync_copy(sa_hbm, sa_v)
        pltpu.sync_copy(da_hbm, da_v)
        pltpu.sync_copy(cnts_hbm, cntv)

        zed = jnp.zeros((LANE,), jnp.float32)

        @plsc.parallel_loop(0, ACCROWS)
        def zrow(r):
            for j in range(HP // LANE):
                acc_t[r, pl.ds(j * LANE, LANE)] = zed

        def writer_body(w, _):
            cw, sw = w // NS, w % NS
            cnt = cntv[cw, sw, pl.ds(b * LANE, LANE)][0]
            nblk = (cnt + 127) // 128

            def blk_body(kb, _):
                off = kb * 128
                pltpu.sync_copy(bsrc_hbm.at[cw, sw, b, pl.ds(off, 128)], srcb)
                pltpu.sync_copy(bdst_hbm.at[cw, sw, b, pl.ds(off, 128)], dstb)

                @plsc.parallel_loop(0, 8)
                def group(g):
                    sl = pl.ds(g * LANE, LANE)
                    dv = dstb[sl]
                    eidx = off + g * LANE + lax.iota(jnp.int32, LANE)
                    m = eidx < cnt
                    sv = jnp.where(m, srcb[sl], 0)
                    srcb[sl] = sv
                    sav = plsc.load_gather(sa_v, [sv])
                    dav = plsc.load_gather(da_v, [jnp.where(m, dv, 0)])
                    xv = sav + dav
                    lv = jnp.where(xv >= 0.0, xv, 0.2 * xv)
                    ev[sl] = jnp.where(m, jnp.exp(lv), 0.0)
                    dstb[sl] = jnp.where(m, dv - base_row, PADROW)
                pltpu.async_copy(h_hbm.at[srcb], grow, sem).wait()

                @plsc.parallel_loop(0, 8)
                def accg(g):
                    evv = ev[pl.ds(g * LANE, LANE)]
                    dlv = dstb[pl.ds(g * LANE, LANE)]
                    for i in range(LANE):
                        er = jnp.full((LANE,), evv[i], jnp.float32)
                        dl = dlv[i]
                        r = g * LANE + i

                        @plsc.parallel_loop(0, H // LANE, unroll=H // LANE)
                        def jloop(j, _er=er, _dl=dl, _r=r):
                            plsc.addupdate(
                                acc_t.at[_dl, pl.ds(j * LANE, LANE)],
                                grow[_r, pl.ds(j * LANE, LANE)] * _er,
                            )

                        plsc.addupdate(acc_t.at[dl, pl.ds(H, LANE)], er)
                return 0

            lax.fori_loop(0, nblk, blk_body, 0)
            return 0

        lax.fori_loop(0, NW, writer_body, 0)

        pltpu.sync_copy(acc_t.at[pl.ds(0, BSIZE)],
                        out_hbm.at[pl.ds(base_row, BSIZE)])

    return k(bsrc, bdst, cnts, h, sa, da)


def kernel(x, edge_index, batch, W1, as1, ad1, b1, W2, as2, ad2, b2,
           W3, as3, ad3, b3, Wp, bp):
    E = edge_index.shape[1]
    e_real = E + N
    e_tot = ((e_real + NW * 128 - 1) // (NW * 128)) * (NW * 128)
    pad = e_tot - e_real
    per_tile = e_tot // NW

    loops = jnp.arange(N, dtype=jnp.int32)
    zpad = jnp.zeros((pad,), jnp.int32)
    src = jnp.concatenate([edge_index[0].astype(jnp.int32), loops, zpad])
    dst = jnp.concatenate([edge_index[1].astype(jnp.int32), loops, zpad])

    bsrc, bdst, cnts = _sc_bin_edges(src, dst, e_real, e_tot)

    a21 = jnp.stack([as1, ad1], axis=1)
    a22 = jnp.stack([as2, ad2], axis=1)
    a23 = jnp.stack([as3, ad3], axis=1)

    h1, sa1, da1 = _tc_first(x.astype(jnp.float32), W1, a21)
    acc1 = _sc_edge_pass(bsrc, bdst, cnts, h1, sa1.reshape(N), da1.reshape(N),
                         per_tile)
    h2, sa2, da2 = _tc_mid(acc1, b1.reshape(1, H), W2, a22)
    acc2 = _sc_edge_pass(bsrc, bdst, cnts, h2, sa2.reshape(N), da2.reshape(N),
                         per_tile)
    h3, sa3, da3 = _tc_mid(acc2, b2.reshape(1, H), W3, a23)
    acc3 = _sc_edge_pass(bsrc, bdst, cnts, h3, sa3.reshape(N), da3.reshape(N),
                         per_tile)

    batf = batch.astype(jnp.int32)
    batm1f = jnp.concatenate([batf[:1], batf[:-1]])
    batch3d = batf.reshape(NGRID, 1, NB)
    batchm13d = batm1f.reshape(NGRID, 1, NB)
    logits = _tc_readout(acc3, b3.reshape(1, H), batch3d, batchm13d,
                         Wp, bp.reshape(1, C))
    return logits


# speedup vs baseline: 3.1957x; 3.1642x over previous
import functools

import jax
import jax.numpy as jnp
from jax import lax
from jax.experimental import pallas as pl
from jax.experimental.pallas import tpu as pltpu
from jax.experimental.pallas import tpu_sc as plsc

N = 10000
D = 128
H = 128
C = 32
G = 64

NC = 2
NS = 16
LANE = 16
NW = NC * NS

NHALF = 5008
NP = 2 * NHALF
ACCROWS = NHALF + 8
TRASH = NHALF
HP = H + 16

NB = 1000
NGRID = N // NB

EPS = 1e-16

_SC_PARAMS = pltpu.CompilerParams(
    needs_layout_passes=False, use_tc_tiling_on_sc=False
)


def _tc_first(x, W, a2):
    def body(x_ref, w_ref, a_ref, h_ref, sa_ref, da_ref):
        h = jnp.dot(x_ref[...], w_ref[...], preferred_element_type=jnp.float32)
        h_ref[...] = h
        sada = jnp.dot(h, a_ref[...], preferred_element_type=jnp.float32)
        sa_ref[...] = sada[:, 0:1]
        da_ref[...] = sada[:, 1:2]

    return pl.pallas_call(
        body,
        grid=(NGRID,),
        in_specs=[
            pl.BlockSpec((NB, D), lambda k: (k, 0)),
            pl.BlockSpec((D, H), lambda k: (0, 0)),
            pl.BlockSpec((H, 2), lambda k: (0, 0)),
        ],
        out_specs=[
            pl.BlockSpec((NB, H), lambda k: (k, 0)),
            pl.BlockSpec((NB, 1), lambda k: (k, 0)),
            pl.BlockSpec((NB, 1), lambda k: (k, 0)),
        ],
        out_shape=[
            jax.ShapeDtypeStruct((N, H), jnp.float32),
            jax.ShapeDtypeStruct((N, 1), jnp.float32),
            jax.ShapeDtypeStruct((N, 1), jnp.float32),
        ],
    )(x, W, a2)


def _tc_mid(acc, b2d, W, a2):
    def body(acc_ref, b_ref, w_ref, a_ref, h_ref, sa_ref, da_ref):
        a0 = acc_ref[...]
        num = a0[:, :H]
        den = a0[:, H:H + 1] + EPS
        feat = jnp.maximum(num / den + b_ref[...], 0.0)
        h = jnp.dot(feat, w_ref[...], preferred_element_type=jnp.float32)
        h_ref[...] = h
        sada = jnp.dot(h, a_ref[...], preferred_element_type=jnp.float32)
        sa_ref[...] = sada[:, 0:1]
        da_ref[...] = sada[:, 1:2]

    return pl.pallas_call(
        body,
        grid=(NGRID,),
        in_specs=[
            pl.BlockSpec((NB, HP), lambda k: (k, 0)),
            pl.BlockSpec((1, H), lambda k: (0, 0)),
            pl.BlockSpec((D, H), lambda k: (0, 0)),
            pl.BlockSpec((H, 2), lambda k: (0, 0)),
        ],
        out_specs=[
            pl.BlockSpec((NB, H), lambda k: (k, 0)),
            pl.BlockSpec((NB, 1), lambda k: (k, 0)),
            pl.BlockSpec((NB, 1), lambda k: (k, 0)),
        ],
        out_shape=[
            jax.ShapeDtypeStruct((N, H), jnp.float32),
            jax.ShapeDtypeStruct((N, 1), jnp.float32),
            jax.ShapeDtypeStruct((N, 1), jnp.float32),
        ],
    )(acc, b2d, W, a2)


def _tc_readout(acc, b2d, batch3d, batchm13d, Wp, bp2d):
    def body(acc_ref, b_ref, bat_ref, batm1_ref, wp_ref, bp_ref, out_ref,
             hsel_ref, pres_ref):
        k = pl.program_id(0)

        a0 = acc_ref[...]
        num = a0[:, :H]
        den = a0[:, H:H + 1] + EPS
        h3 = num / den + b_ref[...]

        bat = bat_ref[0]
        batm1 = batm1_ref[0]
        col = lax.broadcasted_iota(jnp.int32, (1, NB), 1) + k * NB
        first = jnp.logical_or(col == 0, bat != batm1)
        gids = lax.broadcasted_iota(jnp.int32, (G, NB), 0)
        onehot = jnp.where(
            jnp.logical_and(bat == gids, first), 1.0, 0.0
        ).astype(jnp.float32)

        part = jnp.dot(onehot, h3, preferred_element_type=jnp.float32)
        pcnt = jnp.sum(onehot, axis=1, keepdims=True)

        @pl.when(k == 0)
        def _():
            hsel_ref[...] = part
            pres_ref[...] = pcnt

        @pl.when(k > 0)
        def _():
            hsel_ref[...] = hsel_ref[...] + part
            pres_ref[...] = pres_ref[...] + pcnt

        @pl.when(k == NGRID - 1)
        def _():
            lastrow = h3[NB - 1:NB, :]
            hsel = hsel_ref[...] + (1.0 - pres_ref[...]) * lastrow
            out_ref[...] = (
                jnp.dot(hsel, wp_ref[...], preferred_element_type=jnp.float32)
                + bp_ref[...]
            )

    return pl.pallas_call(
        body,
        grid=(NGRID,),
        in_specs=[
            pl.BlockSpec((NB, HP), lambda k: (k, 0)),
            pl.BlockSpec((1, H), lambda k: (0, 0)),
            pl.BlockSpec((1, 1, NB), lambda k: (k, 0, 0)),
            pl.BlockSpec((1, 1, NB), lambda k: (k, 0, 0)),
            pl.BlockSpec((H, C), lambda k: (0, 0)),
            pl.BlockSpec((1, C), lambda k: (0, 0)),
        ],
        out_specs=pl.BlockSpec((G, C), lambda k: (0, 0)),
        out_shape=jax.ShapeDtypeStruct((G, C), jnp.float32),
        scratch_shapes=[
            pltpu.VMEM((G, H), jnp.float32),
            pltpu.VMEM((G, 1), jnp.float32),
        ],
    )(acc, b2d, batch3d, batchm13d, Wp, bp2d)


def _sc_bin_edges(src_pad, dst_pad, e_real, e_tot, capb):
    per_tile = e_tot // NW
    n_chunks = per_tile // 128
    mesh = plsc.VectorSubcoreMesh(core_axis_name="c", subcore_axis_name="s")

    @functools.partial(
        pl.kernel,
        out_type=[
            jax.ShapeDtypeStruct((NC, 2, capb * 128), jnp.int32),
            jax.ShapeDtypeStruct((NC, 2, capb * 128), jnp.int32),
            jax.ShapeDtypeStruct((NC, 2, LANE), jnp.int32),
        ],
        mesh=mesh,
        compiler_params=_SC_PARAMS,
        scratch_types=[
            pltpu.VMEM((128,), jnp.int32),
            pltpu.VMEM((128,), jnp.int32),
            pltpu.VMEM((2, 256), jnp.int32),
            pltpu.VMEM((2, 256), jnp.int32),
            pltpu.VMEM((2, LANE), jnp.int32),
            pltpu.SMEM((2,), jnp.int32),
            pltpu.SMEM((2,), jnp.int32),
        ],
    )
    def k(src_hbm, dst_hbm, bsrc, bdst, cnts,
          srcb, dstb, ssrc, sdst, cv, pcnt, bcnt):
        c = lax.axis_index("c")
        s = lax.axis_index("s")
        wid = c * NS + s
        pcnt[0] = 0
        pcnt[1] = 0

        @pl.when(s == 0)
        def _():
            bcnt[0] = 0
            bcnt[1] = 0

        plsc.subcore_barrier()

        def flush(h):
            slot = plsc.fetch_and_add(bcnt.at[h], 1, subcore_id=0)
            pltpu.sync_copy(ssrc.at[h, pl.ds(0, 128)],
                            bsrc.at[c, h, pl.ds(slot * 128, 128)])
            pltpu.sync_copy(sdst.at[h, pl.ds(0, 128)],
                            bdst.at[c, h, pl.ds(slot * 128, 128)])

        def chunk_body(ch, _):
            base = wid * per_tile + ch * 128
            pltpu.sync_copy(src_hbm.at[pl.ds(base, 128)], srcb)
            pltpu.sync_copy(dst_hbm.at[pl.ds(base, 128)], dstb)

            def group(j, _):
                sl = pl.ds(j * LANE, LANE)
                sv = srcb[sl]
                dv = dstb[sl]
                eid = base + j * LANE + lax.iota(jnp.int32, LANE)
                valid = eid < e_real
                hi = dv >= NHALF
                for h in range(2):
                    mh = hi if h else jnp.logical_not(hi)
                    m = jnp.logical_and(valid, mh)
                    off = pcnt[h]
                    plsc.store_compressed(ssrc.at[h, pl.ds(off, LANE)], sv,
                                          mask=m)
                    plsc.store_compressed(sdst.at[h, pl.ds(off, LANE)], dv,
                                          mask=m)
                    npop = plsc.all_reduce_population_count(m)[0]
                    pcnt[h] = off + npop
                return 0

            lax.fori_loop(0, 8, group, 0)

            for h in range(2):
                sc_h = pcnt[h]

                @pl.when(sc_h >= 128)
                def _(h=h, sc_h=sc_h):
                    flush(h)
                    for q in range(8):
                        ssrc[h, pl.ds(q * LANE, LANE)] = (
                            ssrc[h, pl.ds(128 + q * LANE, LANE)])
                        sdst[h, pl.ds(q * LANE, LANE)] = (
                            sdst[h, pl.ds(128 + q * LANE, LANE)])
                    pcnt[h] = sc_h - 128
            return 0

        lax.fori_loop(0, n_chunks, chunk_body, 0)

        for h in range(2):
            sc_h = pcnt[h]

            @pl.when(sc_h > 0)
            def _(h=h, sc_h=sc_h):
                for q in range(8):
                    sl = pl.ds(q * LANE, LANE)
                    lid = q * LANE + lax.iota(jnp.int32, LANE)
                    mv = lid < sc_h
                    ssrc[h, sl] = jnp.where(mv, ssrc[h, sl], 0)
                    sdst[h, sl] = jnp.where(mv, sdst[h, sl], -1)
                flush(h)

        plsc.subcore_barrier()

        @pl.when(s == 0)
        def _():
            for h in range(2):
                cv[h, pl.ds(0, LANE)] = jnp.full((LANE,), bcnt[h], jnp.int32)
            pltpu.sync_copy(cv, cnts.at[c])

    return k(src_pad, dst_pad)


def _sc_edge_pass(bsrc, bdst, cnts, h, sa, da, capb):
    mesh = plsc.VectorSubcoreMesh(core_axis_name="c", subcore_axis_name="s")
    max_iters = (2 * capb + NS - 1) // NS

    @functools.partial(
        pl.kernel,
        out_type=jax.ShapeDtypeStruct((NP, HP), jnp.float32),
        mesh=mesh,
        compiler_params=_SC_PARAMS,
        scratch_types=[
            pltpu.VMEM((N,), jnp.float32),
            pltpu.VMEM((N,), jnp.float32),
            pltpu.VMEM((NC, 2, LANE), jnp.int32),
            pltpu.VMEM((128,), jnp.int32),
            pltpu.VMEM((128,), jnp.int32),
            pltpu.VMEM((128,), jnp.float32),
            pltpu.VMEM((128, H), jnp.float32),
            pltpu.VMEM((128, HP), jnp.float32),
            pltpu.VMEM_SHARED((ACCROWS, HP), jnp.float32),
            pltpu.SemaphoreType.DMA,
        ],
    )
    def k(bsrc_hbm, bdst_hbm, cnts_hbm, h_hbm, sa_hbm, da_hbm, out_hbm,
          sa_v, da_v, cntv, srcb, dstb, ev, grow, wrow, acc_sh, sem):
        c = lax.axis_index("c")
        s = lax.axis_index("s")
        row0 = c * NHALF

        pltpu.sync_copy(sa_hbm, sa_v)
        pltpu.sync_copy(da_hbm, da_v)
        pltpu.sync_copy(cnts_hbm, cntv)

        zed = jnp.zeros((LANE,), jnp.float32)

        def zrow(r, _):
            for j in range(HP // LANE):
                wrow[r, pl.ds(j * LANE, LANE)] = zed
            return 0

        lax.fori_loop(0, 128, zrow, 0)
        for q in range(3):
            pltpu.sync_copy(wrow.at[pl.ds(0, 104)],
                            acc_sh.at[pl.ds(s * 312 + q * 104, 104)])

        @pl.when(s == 0)
        def _():
            pltpu.sync_copy(wrow.at[pl.ds(0, 24)], acc_sh.at[pl.ds(4992, 24)])

        plsc.subcore_barrier()

        nb0 = cntv[0, c, pl.ds(0, LANE)][0]
        nb1 = cntv[1, c, pl.ds(0, LANE)][0]
        total = nb0 + nb1
        niter = jnp.minimum((total - s + NS - 1) // NS, max_iters)
        niter = jnp.maximum(niter, 0)

        def blk_body(i, _):
            blk = s + i * NS
            wsel = jnp.where(blk < nb0, 0, 1)
            bidx = jnp.where(blk < nb0, blk, blk - nb0)
            pltpu.sync_copy(bsrc_hbm.at[wsel, c, pl.ds(bidx * 128, 128)], srcb)
            pltpu.sync_copy(bdst_hbm.at[wsel, c, pl.ds(bidx * 128, 128)], dstb)

            for j in range(8):
                sl = pl.ds(j * LANE, LANE)
                sv = srcb[sl]
                dv = dstb[sl]
                live = dv >= 0
                sav = plsc.load_gather(sa_v, [sv])
                dav = plsc.load_gather(da_v, [jnp.where(live, dv, 0)])
                xv = sav + dav
                lv = jnp.where(xv >= 0.0, xv, 0.2 * xv)
                ev[sl] = jnp.where(live, jnp.exp(lv), 0.0)
                dstb[sl] = jnp.where(live, dv - row0, TRASH)
            pltpu.async_copy(h_hbm.at[srcb], grow, sem).wait()

            def row_group(g, _):
                evv = ev[pl.ds(g * LANE, LANE)]
                for i2 in range(LANE):
                    er = jnp.full((LANE,), evv[i2], jnp.float32)
                    r = g * LANE + i2
                    for j in range(H // LANE):
                        wrow[r, pl.ds(j * LANE, LANE)] = (
                            grow[r, pl.ds(j * LANE, LANE)] * er
                        )
                    wrow[r, pl.ds(H, LANE)] = er
                return 0

            lax.fori_loop(0, 8, row_group, 0)
            pltpu.sync_copy(wrow, acc_sh.at[dstb], add=True)
            return 0

        lax.fori_loop(0, niter, blk_body, 0)
        plsc.subcore_barrier()

        for q in range(3):
            r0 = s * 312 + q * 104
            pltpu.sync_copy(acc_sh.at[pl.ds(r0, 104)], wrow.at[pl.ds(0, 104)])
            pltpu.sync_copy(wrow.at[pl.ds(0, 104)],
                            out_hbm.at[pl.ds(row0 + r0, 104)])

        @pl.when(s == 0)
        def _():
            pltpu.sync_copy(acc_sh.at[pl.ds(4992, 16)], wrow.at[pl.ds(0, 16)])
            pltpu.sync_copy(wrow.at[pl.ds(0, 16)],
                            out_hbm.at[pl.ds(row0 + 4992, 16)])

    return k(bsrc, bdst, cnts, h, sa, da)


def kernel(x, edge_index, batch, W1, as1, ad1, b1, W2, as2, ad2, b2,
           W3, as3, ad3, b3, Wp, bp):
    E = edge_index.shape[1]
    e_real = E + N
    e_tot = ((e_real + NW * 128 - 1) // (NW * 128)) * (NW * 128)
    pad = e_tot - e_real
    capb = (e_tot // 2) // 128 + NS

    loops = jnp.arange(N, dtype=jnp.int32)
    zpad = jnp.zeros((pad,), jnp.int32)
    src = jnp.concatenate([edge_index[0].astype(jnp.int32), loops, zpad])
    dst = jnp.concatenate([edge_index[1].astype(jnp.int32), loops, zpad])

    bsrc, bdst, cnts = _sc_bin_edges(src, dst, e_real, e_tot, capb)

    a21 = jnp.stack([as1, ad1], axis=1)
    a22 = jnp.stack([as2, ad2], axis=1)
    a23 = jnp.stack([as3, ad3], axis=1)

    h1, sa1, da1 = _tc_first(x.astype(jnp.float32), W1, a21)
    acc1 = _sc_edge_pass(bsrc, bdst, cnts, h1, sa1.reshape(N), da1.reshape(N),
                         capb)
    h2, sa2, da2 = _tc_mid(acc1, b1.reshape(1, H), W2, a22)
    acc2 = _sc_edge_pass(bsrc, bdst, cnts, h2, sa2.reshape(N), da2.reshape(N),
                         capb)
    h3, sa3, da3 = _tc_mid(acc2, b2.reshape(1, H), W3, a23)
    acc3 = _sc_edge_pass(bsrc, bdst, cnts, h3, sa3.reshape(N), da3.reshape(N),
                         capb)

    batf = batch.astype(jnp.int32)
    batm1f = jnp.concatenate([batf[:1], batf[:-1]])
    batch3d = batf.reshape(NGRID, 1, NB)
    batchm13d = batm1f.reshape(NGRID, 1, NB)
    logits = _tc_readout(acc3, b3.reshape(1, H), batch3d, batchm13d,
                         Wp, bp.reshape(1, C))
    return logits


# speedup vs baseline: 3.4225x; 1.0710x over previous
import functools

import jax
import jax.numpy as jnp
from jax import lax
from jax.experimental import pallas as pl
from jax.experimental.pallas import tpu as pltpu
from jax.experimental.pallas import tpu_sc as plsc

N = 10000
D = 128
H = 128
C = 32
G = 64

NC = 2
NS = 16
LANE = 16
NW = NC * NS

NHALF = 5008
NP = 2 * NHALF
ACCROWS = NHALF + 8
TRASH = NHALF
HP = H + 16

NB = 1000
NGRID = N // NB

EPS = 1e-16

_SC_PARAMS = pltpu.CompilerParams(
    needs_layout_passes=False, use_tc_tiling_on_sc=False
)


def _tc_first(x, W, a2):
    def body(x_ref, w_ref, a_ref, h_ref, sa_ref, da_ref):
        h = jnp.dot(x_ref[...], w_ref[...], preferred_element_type=jnp.float32)
        h_ref[...] = h
        sada = jnp.dot(h, a_ref[...], preferred_element_type=jnp.float32)
        sa_ref[...] = sada[:, 0:1]
        da_ref[...] = sada[:, 1:2]

    return pl.pallas_call(
        body,
        grid=(NGRID,),
        in_specs=[
            pl.BlockSpec((NB, D), lambda k: (k, 0)),
            pl.BlockSpec((D, H), lambda k: (0, 0)),
            pl.BlockSpec((H, 2), lambda k: (0, 0)),
        ],
        out_specs=[
            pl.BlockSpec((NB, H), lambda k: (k, 0)),
            pl.BlockSpec((NB, 1), lambda k: (k, 0)),
            pl.BlockSpec((NB, 1), lambda k: (k, 0)),
        ],
        out_shape=[
            jax.ShapeDtypeStruct((N, H), jnp.float32),
            jax.ShapeDtypeStruct((N, 1), jnp.float32),
            jax.ShapeDtypeStruct((N, 1), jnp.float32),
        ],
    )(x, W, a2)


def _tc_mid(acc, b2d, W, a2):
    def body(acc_ref, b_ref, w_ref, a_ref, h_ref, sa_ref, da_ref):
        a0 = acc_ref[...]
        num = a0[:, :H]
        den = a0[:, H:H + 1] + EPS
        feat = jnp.maximum(num / den + b_ref[...], 0.0)
        h = jnp.dot(feat, w_ref[...], preferred_element_type=jnp.float32)
        h_ref[...] = h
        sada = jnp.dot(h, a_ref[...], preferred_element_type=jnp.float32)
        sa_ref[...] = sada[:, 0:1]
        da_ref[...] = sada[:, 1:2]

    return pl.pallas_call(
        body,
        grid=(NGRID,),
        in_specs=[
            pl.BlockSpec((NB, HP), lambda k: (k, 0)),
            pl.BlockSpec((1, H), lambda k: (0, 0)),
            pl.BlockSpec((D, H), lambda k: (0, 0)),
            pl.BlockSpec((H, 2), lambda k: (0, 0)),
        ],
        out_specs=[
            pl.BlockSpec((NB, H), lambda k: (k, 0)),
            pl.BlockSpec((NB, 1), lambda k: (k, 0)),
            pl.BlockSpec((NB, 1), lambda k: (k, 0)),
        ],
        out_shape=[
            jax.ShapeDtypeStruct((N, H), jnp.float32),
            jax.ShapeDtypeStruct((N, 1), jnp.float32),
            jax.ShapeDtypeStruct((N, 1), jnp.float32),
        ],
    )(acc, b2d, W, a2)


def _tc_readout(acc, b2d, batch3d, batchm13d, Wp, bp2d):
    def body(acc_ref, b_ref, bat_ref, batm1_ref, wp_ref, bp_ref, out_ref,
             hsel_ref, pres_ref):
        k = pl.program_id(0)

        a0 = acc_ref[...]
        num = a0[:, :H]
        den = a0[:, H:H + 1] + EPS
        h3 = num / den + b_ref[...]

        bat = bat_ref[0]
        batm1 = batm1_ref[0]
        col = lax.broadcasted_iota(jnp.int32, (1, NB), 1) + k * NB
        first = jnp.logical_or(col == 0, bat != batm1)
        gids = lax.broadcasted_iota(jnp.int32, (G, NB), 0)
        onehot = jnp.where(
            jnp.logical_and(bat == gids, first), 1.0, 0.0
        ).astype(jnp.float32)

        part = jnp.dot(onehot, h3, preferred_element_type=jnp.float32)
        pcnt = jnp.sum(onehot, axis=1, keepdims=True)

        @pl.when(k == 0)
        def _():
            hsel_ref[...] = part
            pres_ref[...] = pcnt

        @pl.when(k > 0)
        def _():
            hsel_ref[...] = hsel_ref[...] + part
            pres_ref[...] = pres_ref[...] + pcnt

        @pl.when(k == NGRID - 1)
        def _():
            lastrow = h3[NB - 1:NB, :]
            hsel = hsel_ref[...] + (1.0 - pres_ref[...]) * lastrow
            out_ref[...] = (
                jnp.dot(hsel, wp_ref[...], preferred_element_type=jnp.float32)
                + bp_ref[...]
            )

    return pl.pallas_call(
        body,
        grid=(NGRID,),
        in_specs=[
            pl.BlockSpec((NB, HP), lambda k: (k, 0)),
            pl.BlockSpec((1, H), lambda k: (0, 0)),
            pl.BlockSpec((1, 1, NB), lambda k: (k, 0, 0)),
            pl.BlockSpec((1, 1, NB), lambda k: (k, 0, 0)),
            pl.BlockSpec((H, C), lambda k: (0, 0)),
            pl.BlockSpec((1, C), lambda k: (0, 0)),
        ],
        out_specs=pl.BlockSpec((G, C), lambda k: (0, 0)),
        out_shape=jax.ShapeDtypeStruct((G, C), jnp.float32),
        scratch_shapes=[
            pltpu.VMEM((G, H), jnp.float32),
            pltpu.VMEM((G, 1), jnp.float32),
        ],
    )(acc, b2d, batch3d, batchm13d, Wp, bp2d)


def _sc_bin_edges(src_pad, dst_pad, e_real, e_tot, capb):
    per_tile = e_tot // NW
    n_chunks = per_tile // 128
    mesh = plsc.VectorSubcoreMesh(core_axis_name="c", subcore_axis_name="s")

    @functools.partial(
        pl.kernel,
        out_type=[
            jax.ShapeDtypeStruct((NC, 2, capb * 128), jnp.int32),
            jax.ShapeDtypeStruct((NC, 2, capb * 128), jnp.int32),
            jax.ShapeDtypeStruct((NC, 2, LANE), jnp.int32),
        ],
        mesh=mesh,
        compiler_params=_SC_PARAMS,
        scratch_types=[
            pltpu.VMEM((128,), jnp.int32),
            pltpu.VMEM((128,), jnp.int32),
            pltpu.VMEM((2, 256), jnp.int32),
            pltpu.VMEM((2, 256), jnp.int32),
            pltpu.VMEM((2, LANE), jnp.int32),
            pltpu.SMEM((2,), jnp.int32),
            pltpu.SMEM((2,), jnp.int32),
        ],
    )
    def k(src_hbm, dst_hbm, bsrc, bdst, cnts,
          srcb, dstb, ssrc, sdst, cv, pcnt, bcnt):
        c = lax.axis_index("c")
        s = lax.axis_index("s")
        wid = c * NS + s
        pcnt[0] = 0
        pcnt[1] = 0

        @pl.when(s == 0)
        def _():
            bcnt[0] = 0
            bcnt[1] = 0

        plsc.subcore_barrier()

        def flush(h):
            slot = plsc.fetch_and_add(bcnt.at[h], 1, subcore_id=0)
            pltpu.sync_copy(ssrc.at[h, pl.ds(0, 128)],
                            bsrc.at[c, h, pl.ds(slot * 128, 128)])
            pltpu.sync_copy(sdst.at[h, pl.ds(0, 128)],
                            bdst.at[c, h, pl.ds(slot * 128, 128)])

        def chunk_body(ch, _):
            base = wid * per_tile + ch * 128
            pltpu.sync_copy(src_hbm.at[pl.ds(base, 128)], srcb)
            pltpu.sync_copy(dst_hbm.at[pl.ds(base, 128)], dstb)

            def group(j, _):
                sl = pl.ds(j * LANE, LANE)
                sv = srcb[sl]
                dv = dstb[sl]
                eid = base + j * LANE + lax.iota(jnp.int32, LANE)
                valid = eid < e_real
                hi = dv >= NHALF
                for h in range(2):
                    mh = hi if h else jnp.logical_not(hi)
                    m = jnp.logical_and(valid, mh)
                    off = pcnt[h]
                    plsc.store_compressed(ssrc.at[h, pl.ds(off, LANE)], sv,
                                          mask=m)
                    plsc.store_compressed(sdst.at[h, pl.ds(off, LANE)], dv,
                                          mask=m)
                    npop = plsc.all_reduce_population_count(m)[0]
                    pcnt[h] = off + npop
                return 0

            lax.fori_loop(0, 8, group, 0)

            for h in range(2):
                sc_h = pcnt[h]

                @pl.when(sc_h >= 128)
                def _(h=h, sc_h=sc_h):
                    flush(h)
                    for q in range(8):
                        ssrc[h, pl.ds(q * LANE, LANE)] = (
                            ssrc[h, pl.ds(128 + q * LANE, LANE)])
                        sdst[h, pl.ds(q * LANE, LANE)] = (
                            sdst[h, pl.ds(128 + q * LANE, LANE)])
                    pcnt[h] = sc_h - 128
            return 0

        lax.fori_loop(0, n_chunks, chunk_body, 0)

        for h in range(2):
            sc_h = pcnt[h]

            @pl.when(sc_h > 0)
            def _(h=h, sc_h=sc_h):
                for q in range(8):
                    sl = pl.ds(q * LANE, LANE)
                    lid = q * LANE + lax.iota(jnp.int32, LANE)
                    mv = lid < sc_h
                    ssrc[h, sl] = jnp.where(mv, ssrc[h, sl], 0)
                    sdst[h, sl] = jnp.where(mv, sdst[h, sl], -1)
                flush(h)

        plsc.subcore_barrier()

        @pl.when(s == 0)
        def _():
            for h in range(2):
                cv[h, pl.ds(0, LANE)] = jnp.full((LANE,), bcnt[h], jnp.int32)
            pltpu.sync_copy(cv, cnts.at[c])

    return k(src_pad, dst_pad)


def _sc_edge_pass(bsrc, bdst, cnts, h, sa, da, capb):
    mesh = plsc.VectorSubcoreMesh(core_axis_name="c", subcore_axis_name="s")
    max_iters = (2 * capb + NS - 1) // NS

    @functools.partial(
        pl.kernel,
        out_type=jax.ShapeDtypeStruct((NP, HP), jnp.float32),
        mesh=mesh,
        compiler_params=_SC_PARAMS,
        scratch_types=[
            pltpu.VMEM((N,), jnp.float32),
            pltpu.VMEM((N,), jnp.float32),
            pltpu.VMEM((NC, 2, LANE), jnp.int32),
            pltpu.VMEM((128,), jnp.int32),
            pltpu.VMEM((2, 128), jnp.int32),
            pltpu.VMEM((128,), jnp.float32),
            pltpu.VMEM((128, H), jnp.float32),
            pltpu.VMEM((2, 128, HP), jnp.float32),
            pltpu.VMEM_SHARED((ACCROWS, HP), jnp.float32),
            pltpu.SemaphoreType.DMA,
            pltpu.SemaphoreType.DMA,
        ],
    )
    def k(bsrc_hbm, bdst_hbm, cnts_hbm, h_hbm, sa_hbm, da_hbm, out_hbm,
          sa_v, da_v, cntv, srcb, dstb2, ev, grow, wrow2, acc_sh, sem, ssem):
        c = lax.axis_index("c")
        s = lax.axis_index("s")
        row0 = c * NHALF

        pltpu.sync_copy(sa_hbm, sa_v)
        pltpu.sync_copy(da_hbm, da_v)
        pltpu.sync_copy(cnts_hbm, cntv)

        zed = jnp.zeros((LANE,), jnp.float32)

        def zrow(r, _):
            for j in range(HP // LANE):
                wrow2[0, r, pl.ds(j * LANE, LANE)] = zed
            return 0

        lax.fori_loop(0, 128, zrow, 0)
        for q in range(3):
            pltpu.sync_copy(wrow2.at[0, pl.ds(0, 104)],
                            acc_sh.at[pl.ds(s * 312 + q * 104, 104)])

        @pl.when(s == 0)
        def _():
            pltpu.sync_copy(wrow2.at[0, pl.ds(0, 24)],
                            acc_sh.at[pl.ds(4992, 24)])

        plsc.subcore_barrier()

        nb0 = cntv[0, c, pl.ds(0, LANE)][0]
        nb1 = cntv[1, c, pl.ds(0, LANE)][0]
        total = nb0 + nb1
        niter = jnp.minimum((total - s + NS - 1) // NS, max_iters)
        niter = jnp.maximum(niter, 0)

        def make_half(p):
            dstb = dstb2.at[p]
            wrow = wrow2.at[p]

            def half(i):
                blk = s + i * NS
                wsel = jnp.where(blk < nb0, 0, 1)
                bidx = jnp.where(blk < nb0, blk, blk - nb0)
                pltpu.sync_copy(bsrc_hbm.at[wsel, c, pl.ds(bidx * 128, 128)],
                                srcb)
                @pl.when(i >= 2)
                def _():
                    pltpu.make_async_copy(wrow, acc_sh.at[dstb], ssem).wait()

                pltpu.sync_copy(bdst_hbm.at[wsel, c, pl.ds(bidx * 128, 128)],
                                dstb)

                for j in range(8):
                    sl = pl.ds(j * LANE, LANE)
                    sv = srcb[sl]
                    dv = dstb[sl]
                    live = dv >= 0
                    sav = plsc.load_gather(sa_v, [sv])
                    dav = plsc.load_gather(da_v, [jnp.where(live, dv, 0)])
                    xv = sav + dav
                    lv = jnp.where(xv >= 0.0, xv, 0.2 * xv)
                    ev[sl] = jnp.where(live, jnp.exp(lv), 0.0)
                    dstb[sl] = jnp.where(live, dv - row0, TRASH)
                pltpu.async_copy(h_hbm.at[srcb], grow, sem).wait()

                def row_group(g, _):
                    evv = ev[pl.ds(g * LANE, LANE)]
                    for i2 in range(LANE):
                        er = jnp.full((LANE,), evv[i2], jnp.float32)
                        r = g * LANE + i2
                        for j in range(H // LANE):
                            wrow[r, pl.ds(j * LANE, LANE)] = (
                                grow[r, pl.ds(j * LANE, LANE)] * er
                            )
                        wrow[r, pl.ds(H, LANE)] = er
                    return 0

                lax.fori_loop(0, 8, row_group, 0)
                pltpu.async_copy(wrow, acc_sh.at[dstb], ssem, add=True)

            return half

        half0 = make_half(0)
        half1 = make_half(1)

        def blk_body(i, _):
            @pl.when(lax.rem(i, 2) == 0)
            def _():
                half0(i)

            @pl.when(lax.rem(i, 2) == 1)
            def _():
                half1(i)
            return 0

        lax.fori_loop(0, niter, blk_body, 0)

        @pl.when(niter >= 1)
        def _():
            pltpu.make_async_copy(wrow2.at[0], acc_sh.at[dstb2.at[0]],
                                  ssem).wait()

        @pl.when(niter >= 2)
        def _():
            pltpu.make_async_copy(wrow2.at[1], acc_sh.at[dstb2.at[1]],
                                  ssem).wait()

        plsc.subcore_barrier()

        for q in range(3):
            r0 = s * 312 + q * 104
            pltpu.sync_copy(acc_sh.at[pl.ds(r0, 104)],
                            wrow2.at[0, pl.ds(0, 104)])
            pltpu.sync_copy(wrow2.at[0, pl.ds(0, 104)],
                            out_hbm.at[pl.ds(row0 + r0, 104)])

        @pl.when(s == 0)
        def _():
            pltpu.sync_copy(acc_sh.at[pl.ds(4992, 16)],
                            wrow2.at[0, pl.ds(0, 16)])
            pltpu.sync_copy(wrow2.at[0, pl.ds(0, 16)],
                            out_hbm.at[pl.ds(row0 + 4992, 16)])

    return k(bsrc, bdst, cnts, h, sa, da)


def kernel(x, edge_index, batch, W1, as1, ad1, b1, W2, as2, ad2, b2,
           W3, as3, ad3, b3, Wp, bp):
    E = edge_index.shape[1]
    e_real = E + N
    e_tot = ((e_real + NW * 128 - 1) // (NW * 128)) * (NW * 128)
    pad = e_tot - e_real
    capb = (e_tot // 2) // 128 + NS

    loops = jnp.arange(N, dtype=jnp.int32)
    zpad = jnp.zeros((pad,), jnp.int32)
    src = jnp.concatenate([edge_index[0].astype(jnp.int32), loops, zpad])
    dst = jnp.concatenate([edge_index[1].astype(jnp.int32), loops, zpad])

    bsrc, bdst, cnts = _sc_bin_edges(src, dst, e_real, e_tot, capb)

    a21 = jnp.stack([as1, ad1], axis=1)
    a22 = jnp.stack([as2, ad2], axis=1)
    a23 = jnp.stack([as3, ad3], axis=1)

    h1, sa1, da1 = _tc_first(x.astype(jnp.float32), W1, a21)
    acc1 = _sc_edge_pass(bsrc, bdst, cnts, h1, sa1.reshape(N), da1.reshape(N),
                         capb)
    h2, sa2, da2 = _tc_mid(acc1, b1.reshape(1, H), W2, a22)
    acc2 = _sc_edge_pass(bsrc, bdst, cnts, h2, sa2.reshape(N), da2.reshape(N),
                         capb)
    h3, sa3, da3 = _tc_mid(acc2, b2.reshape(1, H), W3, a23)
    acc3 = _sc_edge_pass(bsrc, bdst, cnts, h3, sa3.reshape(N), da3.reshape(N),
                         capb)

    batf = batch.astype(jnp.int32)
    batm1f = jnp.concatenate([batf[:1], batf[:-1]])
    batch3d = batf.reshape(NGRID, 1, NB)
    batchm13d = batm1f.reshape(NGRID, 1, NB)
    logits = _tc_readout(acc3, b3.reshape(1, H), batch3d, batchm13d,
                         Wp, bp.reshape(1, C))
    return logits
